# Initial kernel scaffold; baseline (speedup 1.0000x reference)
import functools

import jax
import jax.numpy as jnp
from jax import lax
from jax.experimental import pallas as pl
from jax.experimental.pallas import tpu as pltpu
from jax.experimental.pallas import tpu_sc as plsc

N = 10000
E = 100000
D = 768
NC, NS = 2, 16
CH = 128
NCH = D // CH
BATCH = 128
NB = 50
E_PAD = NS * NB * BATCH
N_PAD = 10240
RPS = N_PAD // NS
DEGW = 16
BN = 400


def _mm_body(x_ref, w_ref, y_ref):
    y_ref[0] = jnp.dot(x_ref[...], w_ref[...],
                       preferred_element_type=jnp.float32)


def _matmul_chunked(x, W):
    return pl.pallas_call(
        _mm_body,
        grid=(N // BN, NCH),
        in_specs=[
            pl.BlockSpec((BN, D), lambda i, c: (i, 0)),
            pl.BlockSpec((D, CH), lambda i, c: (0, c)),
        ],
        out_specs=pl.BlockSpec((1, BN, CH), lambda i, c: (c, i, 0)),
        out_shape=jax.ShapeDtypeStruct((NCH, N, CH), jnp.float32),
    )(x, W)


def _combine_body(y_ref, agg_ref, deg_ref, b_ref, o_ref):
    deg = deg_ref[...][:, 0:1]
    mask = (deg == 0.0).astype(jnp.float32)
    o_ref[...] = jnp.maximum(y_ref[0] * mask + agg_ref[0] + b_ref[0], 0.0)


def _combine(y_t, agg_t, deg, bvec):
    return pl.pallas_call(
        _combine_body,
        grid=(N // BN, NCH),
        in_specs=[
            pl.BlockSpec((1, BN, CH), lambda i, c: (c, i, 0)),
            pl.BlockSpec((1, BN, CH), lambda i, c: (c, i, 0)),
            pl.BlockSpec((BN, DEGW), lambda i, c: (i, 0)),
            pl.BlockSpec((1, CH), lambda i, c: (c, 0)),
        ],
        out_specs=pl.BlockSpec((BN, CH), lambda i, c: (i, c)),
        out_shape=jax.ShapeDtypeStruct((N, D), jnp.float32),
    )(y_t, agg_t, deg, bvec)


_MESH = plsc.VectorSubcoreMesh(core_axis_name="c", subcore_axis_name="s",
                               num_cores=NC, num_subcores=NS)

_AGG_T = jax.ShapeDtypeStruct((NCH, N_PAD, CH), jnp.float32)
_DEG_T = jax.ShapeDtypeStruct((N_PAD, DEGW), jnp.float32)


@functools.partial(
    pl.kernel,
    out_type=[_AGG_T, _AGG_T, _DEG_T, _DEG_T],
    mesh=_MESH,
    scratch_types=[
        pltpu.VMEM((NB, BATCH), jnp.int32),
        pltpu.VMEM((NB, BATCH), jnp.int32),
        pltpu.VMEM((NB, BATCH), jnp.int32),
        pltpu.VMEM((2, BATCH, CH), jnp.float32),
        pltpu.VMEM((BATCH, CH), jnp.float32),
        pltpu.VMEM((BATCH, DEGW), jnp.float32),
        pltpu.VMEM((BATCH, DEGW), jnp.float32),
        pltpu.VMEM_SHARED((N_PAD, CH), jnp.float32),
        pltpu.VMEM_SHARED((N_PAD, DEGW), jnp.float32),
        pltpu.SemaphoreType.DMA,
        pltpu.SemaphoreType.DMA,
    ],
)
def _sc_agg(yl_hbm, yr_hbm, src0_hbm, srcd_hbm, dst0_hbm, dstd_hbm,
            aggl_hbm, aggr_hbm, degl_hbm, degr_hbm,
            gidx_v, sidx_v, didx_v, rows_v, zt_v, zd_v, ones_v,
            acc_sh, deg_sh, sem0, sem1):
    cid = lax.axis_index("c")
    sid = lax.axis_index("s")

    @pl.loop(0, BATCH)
    def _(i):
        @pl.loop(0, CH, step=16)
        def _(j):
            zt_v[i, pl.ds(j, 16)] = jnp.zeros((16,), jnp.float32)

    @pl.loop(0, BATCH)
    def _(i):
        zd_v[i, pl.ds(0, 16)] = jnp.zeros((16,), jnp.float32)
        ones_v[i, pl.ds(0, 16)] = jnp.ones((16,), jnp.float32)

    def run_dir(y_hbm, g_hbm, s_hbm, dg_hbm, agg_hbm, deg_hbm):
        pltpu.sync_copy(g_hbm.at[sid], gidx_v)
        pltpu.sync_copy(s_hbm.at[sid], sidx_v)
        pltpu.sync_copy(dg_hbm.at[sid], didx_v)

        @pl.loop(0, RPS, step=BATCH)
        def _(j):
            pltpu.sync_copy(zd_v, deg_sh.at[pl.ds(sid * RPS + j, BATCH)])
        plsc.subcore_barrier()

        @pl.loop(0, NB)
        def _(b):
            pltpu.sync_copy(ones_v, deg_sh.at[didx_v.at[b]], add=True)
        plsc.subcore_barrier()

        pltpu.sync_copy(deg_sh.at[pl.ds(sid * RPS, RPS)],
                        deg_hbm.at[pl.ds(sid * RPS, RPS)])

        @pl.loop(0, NCH)
        def _(c):
            @pl.loop(0, RPS, step=BATCH)
            def _(j):
                pltpu.sync_copy(zt_v, acc_sh.at[pl.ds(sid * RPS + j, BATCH)])
            plsc.subcore_barrier()

            yc = y_hbm.at[c]

            @pl.loop(0, NB, step=2)
            def _(b):
                cp0 = pltpu.async_copy(yc.at[gidx_v.at[b]], rows_v.at[0], sem0)
                cp1 = pltpu.async_copy(yc.at[gidx_v.at[b + 1]], rows_v.at[1],
                                       sem1)
                cp0.wait()
                pltpu.sync_copy(rows_v.at[0], acc_sh.at[sidx_v.at[b]],
                                add=True)
                cp1.wait()
                pltpu.sync_copy(rows_v.at[1], acc_sh.at[sidx_v.at[b + 1]],
                                add=True)
            plsc.subcore_barrier()

            pltpu.sync_copy(acc_sh.at[pl.ds(sid * RPS, RPS)],
                            agg_hbm.at[c].at[pl.ds(sid * RPS, RPS)])

    @pl.when(cid == 0)
    def _():
        run_dir(yl_hbm, src0_hbm, dstd_hbm, srcd_hbm, aggl_hbm, degl_hbm)

    @pl.when(cid == 1)
    def _():
        run_dir(yr_hbm, dst0_hbm, srcd_hbm, dstd_hbm, aggr_hbm, degr_hbm)


def kernel(x, edge_index, sources, destinations, W_root, b_root, W_leaf,
           b_leaf):
    pad = E_PAD - E
    dummy = (N + (jnp.arange(pad, dtype=jnp.int32) % (N_PAD - N)))
    zpad = jnp.zeros((pad,), jnp.int32)
    shard = (NS, NB, BATCH)
    src0 = jnp.concatenate([sources, zpad]).reshape(shard)
    srcd = jnp.concatenate([sources, dummy]).reshape(shard)
    dst0 = jnp.concatenate([destinations, zpad]).reshape(shard)
    dstd = jnp.concatenate([destinations, dummy]).reshape(shard)

    y_leaf = _matmul_chunked(x, W_leaf)
    y_root = _matmul_chunked(x, W_root)

    agg_l, agg_r, deg_l, deg_r = _sc_agg(y_leaf, y_root,
                                         src0, srcd, dst0, dstd)

    leaf_emb = _combine(y_leaf, agg_l, deg_l, b_leaf.reshape(NCH, CH))
    root_emb = _combine(y_root, agg_r, deg_r, b_root.reshape(NCH, CH))
    return (root_emb, leaf_emb)


# speedup vs baseline: 1.6029x; 1.6029x over previous
import functools

import jax
import jax.numpy as jnp
from jax import lax
from jax.experimental import pallas as pl
from jax.experimental.pallas import tpu as pltpu
from jax.experimental.pallas import tpu_sc as plsc

N = 10000
E = 100000
D = 768
NC, NS = 2, 16
CH = 64
NCH = D // CH
BATCH = 128
NB = 50
E_PAD = NS * NB * BATCH
N_PAD = 10240
RPS = N_PAD // NS
DEGW = 16
BN = 400


def _mm_body(x_ref, w_ref, y_ref):
    res = jnp.dot(x_ref[...], w_ref[...],
                  preferred_element_type=jnp.float32)
    for j in range(NCH):
        y_ref[j] = res[:, j * CH:(j + 1) * CH]


def _matmul_chunked(x, W):
    return pl.pallas_call(
        _mm_body,
        grid=(N // BN,),
        in_specs=[
            pl.BlockSpec((BN, D), lambda i: (i, 0)),
            pl.BlockSpec((D, D), lambda i: (0, 0)),
        ],
        out_specs=pl.BlockSpec((NCH, BN, CH), lambda i: (0, i, 0)),
        out_shape=jax.ShapeDtypeStruct((NCH, N, CH), jnp.float32),
    )(x, W)


def _combine_body(y_ref, agg_ref, deg_ref, b_ref, o_ref):
    deg = deg_ref[...][:, 0:1]
    mask = (deg == 0.0).astype(jnp.float32)
    y = jnp.concatenate([y_ref[0], y_ref[1]], axis=1)
    agg = jnp.concatenate([agg_ref[0], agg_ref[1]], axis=1)
    o_ref[...] = jnp.maximum(y * mask + agg + b_ref[0], 0.0)


def _combine(y_t, agg_t, deg, bvec):
    return pl.pallas_call(
        _combine_body,
        grid=(N // BN, NCH // 2),
        in_specs=[
            pl.BlockSpec((2, BN, CH), lambda i, c: (c, i, 0)),
            pl.BlockSpec((2, BN, CH), lambda i, c: (c, i, 0)),
            pl.BlockSpec((BN, DEGW), lambda i, c: (i, 0)),
            pl.BlockSpec((1, 1, 2 * CH), lambda i, c: (c, 0, 0)),
        ],
        out_specs=pl.BlockSpec((BN, 2 * CH), lambda i, c: (i, c)),
        out_shape=jax.ShapeDtypeStruct((N, D), jnp.float32),
    )(y_t, agg_t, deg, bvec)


_MESH = plsc.VectorSubcoreMesh(core_axis_name="c", subcore_axis_name="s",
                               num_cores=NC, num_subcores=NS)

_AGG_T = jax.ShapeDtypeStruct((NCH, N_PAD, CH), jnp.float32)
_DEG_T = jax.ShapeDtypeStruct((N_PAD, DEGW), jnp.float32)


@functools.partial(
    pl.kernel,
    out_type=[_AGG_T, _AGG_T, _DEG_T, _DEG_T],
    mesh=_MESH,
    compiler_params=pltpu.CompilerParams(use_tc_tiling_on_sc=False),
    scratch_types=[
        pltpu.VMEM((NB, BATCH), jnp.int32),
        pltpu.VMEM((NB, BATCH), jnp.int32),
        pltpu.VMEM((NB, BATCH), jnp.int32),
        pltpu.VMEM((2, BATCH, CH), jnp.float32),
        pltpu.VMEM((BATCH, CH), jnp.float32),
        pltpu.VMEM((BATCH, DEGW), jnp.float32),
        pltpu.VMEM((BATCH, DEGW), jnp.float32),
        pltpu.VMEM_SHARED((N_PAD, CH), jnp.float32),
        pltpu.VMEM_SHARED((N_PAD, DEGW), jnp.float32),
        pltpu.SemaphoreType.DMA,
        pltpu.SemaphoreType.DMA,
    ],
)
def _sc_agg(yl_hbm, yr_hbm, src0_hbm, srcd_hbm, dst0_hbm, dstd_hbm,
            aggl_hbm, aggr_hbm, degl_hbm, degr_hbm,
            gidx_v, sidx_v, didx_v, rows_v, zt_v, zd_v, ones_v,
            acc_sh, deg_sh, sem0, sem1):
    cid = lax.axis_index("c")
    sid = lax.axis_index("s")

    @pl.loop(0, BATCH)
    def _(i):
        @pl.loop(0, CH, step=16)
        def _(j):
            zt_v[i, pl.ds(j, 16)] = jnp.zeros((16,), jnp.float32)

    @pl.loop(0, BATCH)
    def _(i):
        zd_v[i, pl.ds(0, 16)] = jnp.zeros((16,), jnp.float32)
        ones_v[i, pl.ds(0, 16)] = jnp.ones((16,), jnp.float32)

    def run_dir(y_hbm, g_hbm, s_hbm, dg_hbm, agg_hbm, deg_hbm):
        pltpu.sync_copy(g_hbm.at[sid], gidx_v)
        pltpu.sync_copy(s_hbm.at[sid], sidx_v)
        pltpu.sync_copy(dg_hbm.at[sid], didx_v)

        @pl.loop(0, RPS, step=BATCH)
        def _(j):
            pltpu.sync_copy(zd_v, deg_sh.at[pl.ds(sid * RPS + j, BATCH)])
        plsc.subcore_barrier()

        @pl.loop(0, NB)
        def _(b):
            pltpu.sync_copy(ones_v, deg_sh.at[didx_v.at[b]], add=True)
        plsc.subcore_barrier()

        pltpu.sync_copy(deg_sh.at[pl.ds(sid * RPS, RPS)],
                        deg_hbm.at[pl.ds(sid * RPS, RPS)])

        @pl.loop(0, NCH)
        def _(c):
            @pl.loop(0, RPS, step=BATCH)
            def _(j):
                pltpu.sync_copy(zt_v, acc_sh.at[pl.ds(sid * RPS + j, BATCH)])
            plsc.subcore_barrier()

            yc = y_hbm.at[c]

            @pl.loop(0, NB, step=2)
            def _(b):
                cp0 = pltpu.async_copy(yc.at[gidx_v.at[b]], rows_v.at[0], sem0)
                cp1 = pltpu.async_copy(yc.at[gidx_v.at[b + 1]], rows_v.at[1],
                                       sem1)
                cp0.wait()
                pltpu.sync_copy(rows_v.at[0], acc_sh.at[sidx_v.at[b]],
                                add=True)
                cp1.wait()
                pltpu.sync_copy(rows_v.at[1], acc_sh.at[sidx_v.at[b + 1]],
                                add=True)
            plsc.subcore_barrier()

            pltpu.sync_copy(acc_sh.at[pl.ds(sid * RPS, RPS)],
                            agg_hbm.at[c].at[pl.ds(sid * RPS, RPS)])

    @pl.when(cid == 0)
    def _():
        run_dir(yl_hbm, src0_hbm, dstd_hbm, srcd_hbm, aggl_hbm, degl_hbm)

    @pl.when(cid == 1)
    def _():
        run_dir(yr_hbm, dst0_hbm, srcd_hbm, dstd_hbm, aggr_hbm, degr_hbm)


def kernel(x, edge_index, sources, destinations, W_root, b_root, W_leaf,
           b_leaf):
    pad = E_PAD - E
    dummy = (N + (jnp.arange(pad, dtype=jnp.int32) % (N_PAD - N)))
    zpad = jnp.zeros((pad,), jnp.int32)
    shard = (NS, NB, BATCH)
    src0 = jnp.concatenate([sources, zpad]).reshape(shard)
    srcd = jnp.concatenate([sources, dummy]).reshape(shard)
    dst0 = jnp.concatenate([destinations, zpad]).reshape(shard)
    dstd = jnp.concatenate([destinations, dummy]).reshape(shard)

    y_leaf = _matmul_chunked(x, W_leaf)
    y_root = _matmul_chunked(x, W_root)

    agg_l, agg_r, deg_l, deg_r = _sc_agg(y_leaf, y_root,
                                         src0, srcd, dst0, dstd)

    leaf_emb = _combine(y_leaf, agg_l, deg_l, b_leaf.reshape(NCH // 2, 1, 2 * CH))
    root_emb = _combine(y_root, agg_r, deg_r, b_root.reshape(NCH // 2, 1, 2 * CH))
    return (root_emb, leaf_emb)
